# baseline (speedup 1.0000x reference)
import functools

import jax
import jax.numpy as jnp
from jax import lax
from jax.experimental import pallas as pl
from jax.experimental.pallas import tpu as pltpu
from jax.experimental.pallas import tpu_sc as plsc

_NC = 2
_NS = 16
_LANES = 128
_BLK = 400


def _fill2d(ref, nrows, ncols, val):
    v = jnp.full((16,), val, jnp.float32)

    def row(i, carry):
        def col(j, carry2):
            ref[i, pl.ds(j * 16, 16)] = v
            return carry2
        return lax.fori_loop(0, ncols // 16, col, carry)

    lax.fori_loop(0, nrows, row, 0)


def _sc_mesh():
    return plsc.VectorSubcoreMesh(core_axis_name="c", subcore_axis_name="s",
                                  num_cores=_NC, num_subcores=_NS)


def _sc_count(n_pad, rpt):
    rpt_half = rpt // 2
    out_type = [jax.ShapeDtypeStruct((_NC, n_pad, _LANES), jnp.float32)]
    scratch = [
        pltpu.VMEM((rpt_half, _LANES), jnp.int32),
        pltpu.VMEM((_LANES, _LANES), jnp.float32),
        pltpu.VMEM((_LANES, _LANES), jnp.float32),
        pltpu.VMEM_SHARED((n_pad, _LANES), jnp.float32),
    ]
    rows_per_tile = n_pad // _NS

    def body(dst_hbm, cnt_out, dst_v, ones_v, z16_v, cnt_sh):
        c = lax.axis_index("c")
        s = lax.axis_index("s")
        wid = s * _NC + c

        _fill2d(ones_v, _LANES, _LANES, 1.0)
        _fill2d(z16_v, _LANES, _LANES, 0.0)
        base = s * rows_per_tile
        off = 0
        while off < rows_per_tile:
            step = min(_LANES, rows_per_tile - off)
            pltpu.sync_copy(z16_v.at[pl.ds(0, step)],
                            cnt_sh.at[pl.ds(base + off, step)])
            off += step
        plsc.subcore_barrier()

        def step_fn(j, carry):
            pltpu.sync_copy(ones_v, cnt_sh.at[dst_v.at[j]], add=True)
            return carry

        for half in range(2):
            rbase = wid * rpt + half * rpt_half
            pltpu.sync_copy(dst_hbm.at[pl.ds(rbase, rpt_half)], dst_v)
            lax.fori_loop(0, rpt_half, step_fn, 0)

        plsc.subcore_barrier()
        pltpu.sync_copy(cnt_sh.at[pl.ds(base, rows_per_tile)],
                        cnt_out.at[c].at[pl.ds(base, rows_per_tile)])

    return pl.kernel(body, out_type=out_type, mesh=_sc_mesh(),
                     scratch_types=scratch)


def _sc_segsum(n_pad, d, rpt):
    rpt_half = rpt // 2
    out_type = [jax.ShapeDtypeStruct((_NC, n_pad, d), jnp.float32)]
    scratch = [
        pltpu.VMEM((rpt_half, _LANES), jnp.int32),
        pltpu.VMEM((rpt_half, _LANES), jnp.int32),
        pltpu.VMEM((_LANES, d), jnp.float32),
        pltpu.VMEM((_LANES, d), jnp.float32),
        pltpu.VMEM_SHARED((n_pad, d), jnp.float32),
        pltpu.SemaphoreType.DMA,
        pltpu.SemaphoreType.DMA,
        pltpu.SemaphoreType.DMA,
        pltpu.SemaphoreType.DMA,
    ]
    rows_per_tile = n_pad // _NS

    def body(u_hbm, src_hbm, dst_hbm, acc_out, src_v, dst_v, rows0_v,
             rows1_v, acc_sh, sem0, sem1, ssem0, ssem1):
        c = lax.axis_index("c")
        s = lax.axis_index("s")
        wid = s * _NC + c

        _fill2d(rows0_v, _LANES, d, 0.0)
        base = s * rows_per_tile
        off = 0
        while off < rows_per_tile:
            step = min(_LANES, rows_per_tile - off)
            pltpu.sync_copy(rows0_v.at[pl.ds(0, step)],
                            acc_sh.at[pl.ds(base + off, step)])
            off += step
        plsc.subcore_barrier()

        def pair_fn(p, carry):
            j0 = 2 * p
            j1 = j0 + 1

            @pl.when(p > 0)
            def _():
                pltpu.make_async_copy(rows0_v, acc_sh.at[dst_v.at[j0]],
                                      ssem0).wait()
                pltpu.make_async_copy(rows1_v, acc_sh.at[dst_v.at[j1]],
                                      ssem1).wait()

            g0 = pltpu.async_copy(u_hbm.at[src_v.at[j0]], rows0_v, sem0)
            g1 = pltpu.async_copy(u_hbm.at[src_v.at[j1]], rows1_v, sem1)
            g0.wait()
            pltpu.async_copy(rows0_v, acc_sh.at[dst_v.at[j0]], ssem0,
                             add=True)
            g1.wait()
            pltpu.async_copy(rows1_v, acc_sh.at[dst_v.at[j1]], ssem1,
                             add=True)
            return carry

        for half in range(2):
            rbase = wid * rpt + half * rpt_half
            pltpu.sync_copy(src_hbm.at[pl.ds(rbase, rpt_half)], src_v)
            pltpu.sync_copy(dst_hbm.at[pl.ds(rbase, rpt_half)], dst_v)
            lax.fori_loop(0, rpt_half // 2, pair_fn, 0)
            pltpu.make_async_copy(rows0_v, acc_sh.at[dst_v.at[0]],
                                  ssem0).wait()
            pltpu.make_async_copy(rows1_v, acc_sh.at[dst_v.at[1]],
                                  ssem1).wait()

        plsc.subcore_barrier()
        pltpu.sync_copy(acc_sh.at[pl.ds(base, rows_per_tile)],
                        acc_out.at[c].at[pl.ds(base, rows_per_tile)])

    return pl.kernel(body, out_type=out_type, mesh=_sc_mesh(),
                     scratch_types=scratch)


def _tc_pre(x, Wnbr1, Wself1):
    n, d_in = x.shape
    d_h = Wnbr1.shape[1]

    def body(x_ref, wn_ref, ws_ref, u_ref, s_ref):
        xb = x_ref[...]
        u_ref[...] = jnp.dot(xb, wn_ref[...], preferred_element_type=jnp.float32,
                 precision=lax.Precision.HIGHEST)
        s_ref[...] = jnp.dot(xb, ws_ref[...], preferred_element_type=jnp.float32,
                 precision=lax.Precision.HIGHEST)

    return pl.pallas_call(
        body,
        grid=(n // _BLK,),
        in_specs=[pl.BlockSpec((_BLK, d_in), lambda i: (i, 0)),
                  pl.BlockSpec((d_in, d_h), lambda i: (0, 0)),
                  pl.BlockSpec((d_in, d_h), lambda i: (0, 0))],
        out_specs=[pl.BlockSpec((_BLK, d_h), lambda i: (i, 0)),
                   pl.BlockSpec((_BLK, d_h), lambda i: (i, 0))],
        out_shape=[jax.ShapeDtypeStruct((n, d_h), jnp.float32),
                   jax.ShapeDtypeStruct((n, d_h), jnp.float32)],
    )(x, Wnbr1, Wself1)


def _tc_mid(s1, acc1, cnt, b1, Wnbr2, Wself2, b2):
    n, d_h = s1.shape
    n_pad = acc1.shape[1]
    d_emb = Wnbr2.shape[1]

    def body(s1_ref, acc_ref, cnt_ref, b1_ref, wn_ref, ws_ref, b2_ref,
             vhs_ref):
        cnt3 = cnt_ref[...]
        deg = jnp.maximum(cnt3[0, :, :1] + cnt3[1, :, :1], 1.0)
        agg = (acc_ref[0] + acc_ref[1]) / deg
        h = jnp.maximum(s1_ref[...] + agg + b1_ref[...], 0.0)
        v = jnp.dot(h, wn_ref[...], preferred_element_type=jnp.float32,
                 precision=lax.Precision.HIGHEST)
        hs = jnp.dot(h, ws_ref[...],
                     preferred_element_type=jnp.float32,
                 precision=lax.Precision.HIGHEST) + b2_ref[...]
        vhs_ref[...] = jnp.concatenate([v, hs], axis=1)

    return pl.pallas_call(
        body,
        grid=(n // _BLK,),
        in_specs=[pl.BlockSpec((_BLK, d_h), lambda i: (i, 0)),
                  pl.BlockSpec((_NC, _BLK, d_h), lambda i: (0, i, 0)),
                  pl.BlockSpec((_NC, _BLK, _LANES), lambda i: (0, i, 0)),
                  pl.BlockSpec((1, d_h), lambda i: (0, 0)),
                  pl.BlockSpec((d_h, d_emb), lambda i: (0, 0)),
                  pl.BlockSpec((d_h, d_emb), lambda i: (0, 0)),
                  pl.BlockSpec((1, d_emb), lambda i: (0, 0))],
        out_specs=[pl.BlockSpec((_BLK, 2 * d_emb), lambda i: (i, 0))],
        out_shape=[jax.ShapeDtypeStruct((n, 2 * d_emb), jnp.float32)],
    )(s1, acc1, cnt, b1, Wnbr2, Wself2, b2)


def _tc_rep(vhs, acc2, cnt, Wd, bd):
    n = vhs.shape[0]
    d_emb = vhs.shape[1] // 2
    d_dec = Wd.shape[1]

    def body(vhs_ref, acc_ref, cnt_ref, wd_ref, bd_ref, rep_ref, st_ref, sacc):
        i = pl.program_id(0)
        cnt3 = cnt_ref[...]
        deg = jnp.maximum(cnt3[0, :, :1] + cnt3[1, :, :1], 1.0)
        hs = vhs_ref[...][:, d_emb:]
        aggv = (acc_ref[0] + acc_ref[1])[:, :d_emb]
        rep = hs + aggv / deg
        rep_ref[...] = rep
        hd = jnp.dot(rep, wd_ref[...],
                     preferred_element_type=jnp.float32,
                 precision=lax.Precision.HIGHEST) + bd_ref[...]

        @pl.when(i == 0)
        def _():
            sacc[...] = jnp.zeros_like(sacc)

        sacc[0:1, :] += jnp.sum(hd, axis=0, keepdims=True)
        sacc[1:2, :] += jnp.sum(hd * hd, axis=0, keepdims=True)
        st_ref[...] = sacc[...]

    return pl.pallas_call(
        body,
        grid=(n // _BLK,),
        in_specs=[pl.BlockSpec((_BLK, 2 * d_emb), lambda i: (i, 0)),
                  pl.BlockSpec((_NC, _BLK, 2 * d_emb), lambda i: (0, i, 0)),
                  pl.BlockSpec((_NC, _BLK, _LANES), lambda i: (0, i, 0)),
                  pl.BlockSpec((d_emb, d_dec), lambda i: (0, 0)),
                  pl.BlockSpec((1, d_dec), lambda i: (0, 0))],
        out_specs=[pl.BlockSpec((_BLK, d_emb), lambda i: (i, 0)),
                   pl.BlockSpec((2, d_dec), lambda i: (0, 0))],
        out_shape=[jax.ShapeDtypeStruct((n, d_emb), jnp.float32),
                   jax.ShapeDtypeStruct((2, d_dec), jnp.float32)],
        scratch_shapes=[pltpu.VMEM((2, d_dec), jnp.float32)],
    )(vhs, acc2, cnt, Wd, bd)


def _tc_heads(rep, stats, Wd, bd, gamma, beta, Wm, bm, Wdi, bdi, Wpi, bpi):
    n, d_emb = rep.shape
    d_dec = Wd.shape[1]
    g = Wm.shape[1]
    n_f = float(n)

    def body(rep_ref, st_ref, wd_ref, bd_ref, g_ref, be_ref, wm_ref, bm_ref,
             wdi_ref, bdi_ref, wpi_ref, bpi_ref, mean_ref, disp_ref, pi_ref):
        rep = rep_ref[...]
        hd = jnp.dot(rep, wd_ref[...],
                     preferred_element_type=jnp.float32,
                 precision=lax.Precision.HIGHEST) + bd_ref[...]
        st = st_ref[...]
        mu = st[0:1, :] / n_f
        var = st[1:2, :] / n_f - mu * mu
        inv = lax.rsqrt(var + 1e-5)
        hidden = jnp.maximum((hd - mu) * inv * g_ref[...] + be_ref[...], 0.0)
        m = jnp.dot(hidden, wm_ref[...],
                    preferred_element_type=jnp.float32) + bm_ref[...]
        mean_ref[...] = jnp.clip(jnp.exp(m), 1e-5, 1e6)
        dv = jnp.dot(hidden, wdi_ref[...],
                     preferred_element_type=jnp.float32) + bdi_ref[...]
        sp = jnp.maximum(dv, 0.0) + jnp.log(1.0 + jnp.exp(-jnp.abs(dv)))
        disp_ref[...] = jnp.clip(sp, 1e-4, 1e4)
        pv = jnp.dot(hidden, wpi_ref[...],
                     preferred_element_type=jnp.float32) + bpi_ref[...]
        pi_ref[...] = 1.0 / (1.0 + jnp.exp(-pv))

    return pl.pallas_call(
        body,
        grid=(n // _BLK,),
        in_specs=[pl.BlockSpec((_BLK, d_emb), lambda i: (i, 0)),
                  pl.BlockSpec((2, d_dec), lambda i: (0, 0)),
                  pl.BlockSpec((d_emb, d_dec), lambda i: (0, 0)),
                  pl.BlockSpec((1, d_dec), lambda i: (0, 0)),
                  pl.BlockSpec((1, d_dec), lambda i: (0, 0)),
                  pl.BlockSpec((1, d_dec), lambda i: (0, 0)),
                  pl.BlockSpec((d_dec, g), lambda i: (0, 0)),
                  pl.BlockSpec((1, g), lambda i: (0, 0)),
                  pl.BlockSpec((d_dec, g), lambda i: (0, 0)),
                  pl.BlockSpec((1, g), lambda i: (0, 0)),
                  pl.BlockSpec((d_dec, g), lambda i: (0, 0)),
                  pl.BlockSpec((1, g), lambda i: (0, 0))],
        out_specs=[pl.BlockSpec((_BLK, g), lambda i: (i, 0)),
                   pl.BlockSpec((_BLK, g), lambda i: (i, 0)),
                   pl.BlockSpec((_BLK, g), lambda i: (i, 0))],
        out_shape=[jax.ShapeDtypeStruct((n, g), jnp.float32),
                   jax.ShapeDtypeStruct((n, g), jnp.float32),
                   jax.ShapeDtypeStruct((n, g), jnp.float32)],
    )(rep, stats, Wd, bd, gamma, beta, Wm, bm, Wdi, bdi, Wpi, bpi)


def kernel(x, edge_index, Wself1, Wnbr1, b1, Wself2, Wnbr2, b2, Wd, bd,
           gamma, beta, Wm, bm, Wdi, bdi, Wpi, bpi):
    n = x.shape[0]
    e = edge_index.shape[1]
    d_h = Wself1.shape[1]
    d_emb = Wself2.shape[1]

    rpt = -(-e // (_LANES * _NC * _NS))
    rpt = (rpt + 15) // 16 * 16
    rpad = rpt * _NC * _NS
    epad = rpad * _LANES
    n_pad = (n + 128) // 128 * 128

    src = jnp.concatenate(
        [edge_index[0], jnp.zeros((epad - e,), jnp.int32)]).reshape(rpad, _LANES)
    dst = jnp.concatenate(
        [edge_index[1], jnp.full((epad - e,), n, jnp.int32)]).reshape(rpad, _LANES)

    cnt = _sc_count(n_pad, rpt)(dst)
    if isinstance(cnt, (list, tuple)):
        cnt = cnt[0]
    u, s1 = _tc_pre(x, Wnbr1, Wself1)
    acc1 = _sc_segsum(n_pad, d_h, rpt)(u, src, dst)
    if isinstance(acc1, (list, tuple)):
        acc1 = acc1[0]
    vhs = _tc_mid(s1, acc1, cnt, b1.reshape(1, -1), Wnbr2, Wself2,
                  b2.reshape(1, -1))
    if isinstance(vhs, (list, tuple)):
        vhs = vhs[0]
    acc2 = _sc_segsum(n_pad, 2 * d_emb, rpt)(vhs, src, dst)
    if isinstance(acc2, (list, tuple)):
        acc2 = acc2[0]
    rep, stats = _tc_rep(vhs, acc2, cnt, Wd, bd.reshape(1, -1))
    mean, disp, pi = _tc_heads(rep, stats, Wd, bd.reshape(1, -1),
                               gamma.reshape(1, -1), beta.reshape(1, -1),
                               Wm, bm.reshape(1, -1), Wdi, bdi.reshape(1, -1),
                               Wpi, bpi.reshape(1, -1))
    return (mean, disp, pi, rep)


# speedup vs baseline: 1.0129x; 1.0129x over previous
import functools

import jax
import jax.numpy as jnp
from jax import lax
from jax.experimental import pallas as pl
from jax.experimental.pallas import tpu as pltpu
from jax.experimental.pallas import tpu_sc as plsc

_NC = 2
_NS = 16
_LANES = 128
_BLK = 400


def _fill2d(ref, nrows, ncols, val):
    v = jnp.full((16,), val, jnp.float32)

    def row(i, carry):
        def col(j, carry2):
            ref[i, pl.ds(j * 16, 16)] = v
            return carry2
        return lax.fori_loop(0, ncols // 16, col, carry)

    lax.fori_loop(0, nrows, row, 0)


def _sc_mesh():
    return plsc.VectorSubcoreMesh(core_axis_name="c", subcore_axis_name="s",
                                  num_cores=_NC, num_subcores=_NS)


def _sc_count(n_pad, rpt):
    rpt_half = rpt // 2
    out_type = [jax.ShapeDtypeStruct((_NC, n_pad, _LANES), jnp.float32)]
    scratch = [
        pltpu.VMEM((rpt_half, _LANES), jnp.int32),
        pltpu.VMEM((_LANES, _LANES), jnp.float32),
        pltpu.VMEM((_LANES, _LANES), jnp.float32),
        pltpu.VMEM_SHARED((n_pad, _LANES), jnp.float32),
    ]
    rows_per_tile = n_pad // _NS

    def body(dst_hbm, cnt_out, dst_v, ones_v, z16_v, cnt_sh):
        c = lax.axis_index("c")
        s = lax.axis_index("s")
        wid = s * _NC + c

        _fill2d(ones_v, _LANES, _LANES, 1.0)
        _fill2d(z16_v, _LANES, _LANES, 0.0)
        base = s * rows_per_tile
        off = 0
        while off < rows_per_tile:
            step = min(_LANES, rows_per_tile - off)
            pltpu.sync_copy(z16_v.at[pl.ds(0, step)],
                            cnt_sh.at[pl.ds(base + off, step)])
            off += step
        plsc.subcore_barrier()

        def step_fn(j, carry):
            pltpu.sync_copy(ones_v, cnt_sh.at[dst_v.at[j]], add=True)
            return carry

        for half in range(2):
            rbase = wid * rpt + half * rpt_half
            pltpu.sync_copy(dst_hbm.at[pl.ds(rbase, rpt_half)], dst_v)
            lax.fori_loop(0, rpt_half, step_fn, 0)

        plsc.subcore_barrier()
        pltpu.sync_copy(cnt_sh.at[pl.ds(base, rows_per_tile)],
                        cnt_out.at[c].at[pl.ds(base, rows_per_tile)])

    return pl.kernel(body, out_type=out_type, mesh=_sc_mesh(),
                     scratch_types=scratch)


def _sc_segsum(n_pad, d, rpt):
    rpt_half = rpt // 2
    out_type = [jax.ShapeDtypeStruct((_NC, n_pad, d), jnp.float32)]
    scratch = [
        pltpu.VMEM((rpt_half, _LANES), jnp.int32),
        pltpu.VMEM((rpt_half, _LANES), jnp.int32),
        pltpu.VMEM((_LANES, d), jnp.float32),
        pltpu.VMEM((_LANES, d), jnp.float32),
        pltpu.VMEM_SHARED((n_pad, d), jnp.float32),
        pltpu.SemaphoreType.DMA,
        pltpu.SemaphoreType.DMA,
        pltpu.SemaphoreType.DMA,
        pltpu.SemaphoreType.DMA,
    ]
    rows_per_tile = n_pad // _NS

    def body(u_hbm, src_hbm, dst_hbm, acc_out, src_v, dst_v, rows0_v,
             rows1_v, acc_sh, sem0, sem1, ssem0, ssem1):
        c = lax.axis_index("c")
        s = lax.axis_index("s")
        wid = s * _NC + c

        _fill2d(rows0_v, _LANES, d, 0.0)
        base = s * rows_per_tile
        off = 0
        while off < rows_per_tile:
            step = min(_LANES, rows_per_tile - off)
            pltpu.sync_copy(rows0_v.at[pl.ds(0, step)],
                            acc_sh.at[pl.ds(base + off, step)])
            off += step
        plsc.subcore_barrier()

        def pair_fn(p, carry):
            j0 = 2 * p
            j1 = j0 + 1

            @pl.when(p > 0)
            def _():
                pltpu.make_async_copy(rows0_v, acc_sh.at[dst_v.at[j0]],
                                      ssem0).wait()
                pltpu.make_async_copy(rows1_v, acc_sh.at[dst_v.at[j1]],
                                      ssem1).wait()

            g0 = pltpu.async_copy(u_hbm.at[src_v.at[j0]], rows0_v, sem0)
            g1 = pltpu.async_copy(u_hbm.at[src_v.at[j1]], rows1_v, sem1)
            g0.wait()
            pltpu.async_copy(rows0_v, acc_sh.at[dst_v.at[j0]], ssem0,
                             add=True)
            g1.wait()
            pltpu.async_copy(rows1_v, acc_sh.at[dst_v.at[j1]], ssem1,
                             add=True)
            return carry

        for half in range(2):
            rbase = wid * rpt + half * rpt_half
            pltpu.sync_copy(src_hbm.at[pl.ds(rbase, rpt_half)], src_v)
            pltpu.sync_copy(dst_hbm.at[pl.ds(rbase, rpt_half)], dst_v)
            lax.fori_loop(0, rpt_half // 2, pair_fn, 0)
            pltpu.make_async_copy(rows0_v, acc_sh.at[dst_v.at[0]],
                                  ssem0).wait()
            pltpu.make_async_copy(rows1_v, acc_sh.at[dst_v.at[1]],
                                  ssem1).wait()

        plsc.subcore_barrier()
        pltpu.sync_copy(acc_sh.at[pl.ds(base, rows_per_tile)],
                        acc_out.at[c].at[pl.ds(base, rows_per_tile)])

    return pl.kernel(body, out_type=out_type, mesh=_sc_mesh(),
                     scratch_types=scratch)


def _tc_pre(x, Wnbr1, Wself1):
    n, d_in = x.shape
    d_h = Wnbr1.shape[1]

    def body(x_ref, wn_ref, ws_ref, u_ref, s_ref):
        xb = x_ref[...]
        u_ref[...] = jnp.dot(xb, wn_ref[...], preferred_element_type=jnp.float32,
                 precision=lax.Precision.HIGHEST)
        s_ref[...] = jnp.dot(xb, ws_ref[...], preferred_element_type=jnp.float32,
                 precision=lax.Precision.HIGHEST)

    return pl.pallas_call(
        body,
        grid=(n // _BLK,),
        in_specs=[pl.BlockSpec((_BLK, d_in), lambda i: (i, 0)),
                  pl.BlockSpec((d_in, d_h), lambda i: (0, 0)),
                  pl.BlockSpec((d_in, d_h), lambda i: (0, 0))],
        out_specs=[pl.BlockSpec((_BLK, d_h), lambda i: (i, 0)),
                   pl.BlockSpec((_BLK, d_h), lambda i: (i, 0))],
        out_shape=[jax.ShapeDtypeStruct((n, d_h), jnp.float32),
                   jax.ShapeDtypeStruct((n, d_h), jnp.float32)],
    )(x, Wnbr1, Wself1)


def _tc_mid(s1, acc1, cnt, b1, Wnbr2, Wself2, b2):
    n, d_h = s1.shape
    n_pad = acc1.shape[1]
    d_emb = Wnbr2.shape[1]

    def body(s1_ref, acc_ref, cnt_ref, b1_ref, wn_ref, ws_ref, b2_ref,
             vhs_ref):
        cnt3 = cnt_ref[...]
        deg = jnp.maximum(cnt3[0, :, :1] + cnt3[1, :, :1], 1.0)
        agg = (acc_ref[0] + acc_ref[1]) / deg
        h = jnp.maximum(s1_ref[...] + agg + b1_ref[...], 0.0)
        v = jnp.dot(h, wn_ref[...], preferred_element_type=jnp.float32,
                 precision=lax.Precision.HIGHEST)
        hs = jnp.dot(h, ws_ref[...],
                     preferred_element_type=jnp.float32,
                 precision=lax.Precision.HIGHEST) + b2_ref[...]
        vhs_ref[...] = jnp.concatenate([v, hs], axis=1)

    return pl.pallas_call(
        body,
        grid=(n // _BLK,),
        in_specs=[pl.BlockSpec((_BLK, d_h), lambda i: (i, 0)),
                  pl.BlockSpec((_NC, _BLK, d_h), lambda i: (0, i, 0)),
                  pl.BlockSpec((_NC, _BLK, _LANES), lambda i: (0, i, 0)),
                  pl.BlockSpec((1, d_h), lambda i: (0, 0)),
                  pl.BlockSpec((d_h, d_emb), lambda i: (0, 0)),
                  pl.BlockSpec((d_h, d_emb), lambda i: (0, 0)),
                  pl.BlockSpec((1, d_emb), lambda i: (0, 0))],
        out_specs=[pl.BlockSpec((_BLK, 2 * d_emb), lambda i: (i, 0))],
        out_shape=[jax.ShapeDtypeStruct((n, 2 * d_emb), jnp.float32)],
    )(s1, acc1, cnt, b1, Wnbr2, Wself2, b2)


def _tc_rep(vhs, acc2, cnt, Wd, bd):
    n = vhs.shape[0]
    d_emb = vhs.shape[1] // 2
    d_dec = Wd.shape[1]

    def body(vhs_ref, acc_ref, cnt_ref, wd_ref, bd_ref, rep_ref, st_ref, sacc):
        i = pl.program_id(0)
        cnt3 = cnt_ref[...]
        deg = jnp.maximum(cnt3[0, :, :1] + cnt3[1, :, :1], 1.0)
        hs = vhs_ref[...][:, d_emb:]
        aggv = (acc_ref[0] + acc_ref[1])[:, :d_emb]
        rep = hs + aggv / deg
        rep_ref[...] = rep
        hd = jnp.dot(rep, wd_ref[...],
                     preferred_element_type=jnp.float32,
                 precision=lax.Precision.HIGHEST) + bd_ref[...]

        @pl.when(i == 0)
        def _():
            sacc[...] = jnp.zeros_like(sacc)

        sacc[0:1, :] += jnp.sum(hd, axis=0, keepdims=True)
        sacc[1:2, :] += jnp.sum(hd * hd, axis=0, keepdims=True)
        st_ref[...] = sacc[...]

    return pl.pallas_call(
        body,
        grid=(n // _BLK,),
        in_specs=[pl.BlockSpec((_BLK, 2 * d_emb), lambda i: (i, 0)),
                  pl.BlockSpec((_NC, _BLK, 2 * d_emb), lambda i: (0, i, 0)),
                  pl.BlockSpec((_NC, _BLK, _LANES), lambda i: (0, i, 0)),
                  pl.BlockSpec((d_emb, d_dec), lambda i: (0, 0)),
                  pl.BlockSpec((1, d_dec), lambda i: (0, 0))],
        out_specs=[pl.BlockSpec((_BLK, d_emb), lambda i: (i, 0)),
                   pl.BlockSpec((2, d_dec), lambda i: (0, 0))],
        out_shape=[jax.ShapeDtypeStruct((n, d_emb), jnp.float32),
                   jax.ShapeDtypeStruct((2, d_dec), jnp.float32)],
        scratch_shapes=[pltpu.VMEM((2, d_dec), jnp.float32)],
    )(vhs, acc2, cnt, Wd, bd)


def _tc_heads(rep, stats, Wd, bd, gamma, beta, Wm, bm, Wdi, bdi, Wpi, bpi):
    n, d_emb = rep.shape
    d_dec = Wd.shape[1]
    g = Wm.shape[1]
    n_f = float(n)
    blk = 1000

    def body(rep_ref, st_ref, wd_ref, bd_ref, g_ref, be_ref, wm_ref, bm_ref,
             wdi_ref, bdi_ref, wpi_ref, bpi_ref, mean_ref, disp_ref, pi_ref):
        rep = rep_ref[...]
        hd = jnp.dot(rep, wd_ref[...],
                     preferred_element_type=jnp.float32,
                 precision=lax.Precision.HIGHEST) + bd_ref[...]
        st = st_ref[...]
        mu = st[0:1, :] / n_f
        var = st[1:2, :] / n_f - mu * mu
        inv = lax.rsqrt(var + 1e-5)
        hidden = jnp.maximum((hd - mu) * inv * g_ref[...] + be_ref[...], 0.0)
        m = jnp.dot(hidden, wm_ref[...],
                    preferred_element_type=jnp.float32) + bm_ref[...]
        mean_ref[...] = jnp.clip(jnp.exp(m), 1e-5, 1e6)
        dv = jnp.dot(hidden, wdi_ref[...],
                     preferred_element_type=jnp.float32) + bdi_ref[...]
        sp = jnp.maximum(dv, 0.0) + jnp.log(1.0 + jnp.exp(-jnp.abs(dv)))
        disp_ref[...] = jnp.clip(sp, 1e-4, 1e4)
        pv = jnp.dot(hidden, wpi_ref[...],
                     preferred_element_type=jnp.float32) + bpi_ref[...]
        pi_ref[...] = 1.0 / (1.0 + jnp.exp(-pv))

    return pl.pallas_call(
        body,
        grid=(n // blk,),
        in_specs=[pl.BlockSpec((blk, d_emb), lambda i: (i, 0)),
                  pl.BlockSpec((2, d_dec), lambda i: (0, 0)),
                  pl.BlockSpec((d_emb, d_dec), lambda i: (0, 0)),
                  pl.BlockSpec((1, d_dec), lambda i: (0, 0)),
                  pl.BlockSpec((1, d_dec), lambda i: (0, 0)),
                  pl.BlockSpec((1, d_dec), lambda i: (0, 0)),
                  pl.BlockSpec((d_dec, g), lambda i: (0, 0)),
                  pl.BlockSpec((1, g), lambda i: (0, 0)),
                  pl.BlockSpec((d_dec, g), lambda i: (0, 0)),
                  pl.BlockSpec((1, g), lambda i: (0, 0)),
                  pl.BlockSpec((d_dec, g), lambda i: (0, 0)),
                  pl.BlockSpec((1, g), lambda i: (0, 0))],
        out_specs=[pl.BlockSpec((blk, g), lambda i: (i, 0)),
                   pl.BlockSpec((blk, g), lambda i: (i, 0)),
                   pl.BlockSpec((blk, g), lambda i: (i, 0))],
        out_shape=[jax.ShapeDtypeStruct((n, g), jnp.float32),
                   jax.ShapeDtypeStruct((n, g), jnp.float32),
                   jax.ShapeDtypeStruct((n, g), jnp.float32)],
    )(rep, stats, Wd, bd, gamma, beta, Wm, bm, Wdi, bdi, Wpi, bpi)


def kernel(x, edge_index, Wself1, Wnbr1, b1, Wself2, Wnbr2, b2, Wd, bd,
           gamma, beta, Wm, bm, Wdi, bdi, Wpi, bpi):
    n = x.shape[0]
    e = edge_index.shape[1]
    d_h = Wself1.shape[1]
    d_emb = Wself2.shape[1]

    rpt = -(-e // (_LANES * _NC * _NS))
    rpt = (rpt + 15) // 16 * 16
    rpad = rpt * _NC * _NS
    epad = rpad * _LANES
    n_pad = (n + 128) // 128 * 128

    src = jnp.concatenate(
        [edge_index[0], jnp.zeros((epad - e,), jnp.int32)]).reshape(rpad, _LANES)
    dst = jnp.concatenate(
        [edge_index[1], jnp.full((epad - e,), n, jnp.int32)]).reshape(rpad, _LANES)

    cnt = _sc_count(n_pad, rpt)(dst)
    if isinstance(cnt, (list, tuple)):
        cnt = cnt[0]
    u, s1 = _tc_pre(x, Wnbr1, Wself1)
    acc1 = _sc_segsum(n_pad, d_h, rpt)(u, src, dst)
    if isinstance(acc1, (list, tuple)):
        acc1 = acc1[0]
    vhs = _tc_mid(s1, acc1, cnt, b1.reshape(1, -1), Wnbr2, Wself2,
                  b2.reshape(1, -1))
    if isinstance(vhs, (list, tuple)):
        vhs = vhs[0]
    acc2 = _sc_segsum(n_pad, 2 * d_emb, rpt)(vhs, src, dst)
    if isinstance(acc2, (list, tuple)):
        acc2 = acc2[0]
    rep, stats = _tc_rep(vhs, acc2, cnt, Wd, bd.reshape(1, -1))
    mean, disp, pi = _tc_heads(rep, stats, Wd, bd.reshape(1, -1),
                               gamma.reshape(1, -1), beta.reshape(1, -1),
                               Wm, bm.reshape(1, -1), Wdi, bdi.reshape(1, -1),
                               Wpi, bpi.reshape(1, -1))
    return (mean, disp, pi, rep)


# speedup vs baseline: 1.0299x; 1.0168x over previous
import jax
import jax.numpy as jnp
from jax import lax
from jax.experimental import pallas as pl
from jax.experimental.pallas import tpu as pltpu
from jax.experimental.pallas import tpu_sc as plsc

_NC = 2
_NS = 16
_LANES = 128
_BLK = 400


def _fill2d(ref, nrows, ncols, val):
    v = jnp.full((16,), val, jnp.float32)

    def row(i, carry):
        def col(j, carry2):
            ref[i, pl.ds(j * 16, 16)] = v
            return carry2
        return lax.fori_loop(0, ncols // 16, col, carry)

    lax.fori_loop(0, nrows, row, 0)


def _sc_mesh():
    return plsc.VectorSubcoreMesh(core_axis_name="c", subcore_axis_name="s",
                                  num_cores=_NC, num_subcores=_NS)


def _sc_count(n_pad, rpt):
    rpt_half = rpt // 2
    out_type = [jax.ShapeDtypeStruct((_NC, n_pad, _LANES), jnp.float32)]
    scratch = [
        pltpu.VMEM((rpt_half, _LANES), jnp.int32),
        pltpu.VMEM((_LANES, _LANES), jnp.float32),
        pltpu.VMEM((_LANES, _LANES), jnp.float32),
        pltpu.VMEM_SHARED((n_pad, _LANES), jnp.float32),
    ]
    rows_per_tile = n_pad // _NS

    def body(dst_hbm, cnt_out, dst_v, ones_v, z16_v, cnt_sh):
        c = lax.axis_index("c")
        s = lax.axis_index("s")
        wid = s * _NC + c

        _fill2d(ones_v, _LANES, _LANES, 1.0)
        _fill2d(z16_v, _LANES, _LANES, 0.0)
        base = s * rows_per_tile
        off = 0
        while off < rows_per_tile:
            step = min(_LANES, rows_per_tile - off)
            pltpu.sync_copy(z16_v.at[pl.ds(0, step)],
                            cnt_sh.at[pl.ds(base + off, step)])
            off += step
        plsc.subcore_barrier()

        def step_fn(j, carry):
            pltpu.sync_copy(ones_v, cnt_sh.at[dst_v.at[j]], add=True)
            return carry

        for half in range(2):
            rbase = wid * rpt + half * rpt_half
            pltpu.sync_copy(dst_hbm.at[pl.ds(rbase, rpt_half)], dst_v)
            lax.fori_loop(0, rpt_half, step_fn, 0)

        plsc.subcore_barrier()
        pltpu.sync_copy(cnt_sh.at[pl.ds(base, rows_per_tile)],
                        cnt_out.at[c].at[pl.ds(base, rows_per_tile)])

    return pl.kernel(body, out_type=out_type, mesh=_sc_mesh(),
                     scratch_types=scratch)


def _sc_segsum(n_pad, d, rpt):
    rpt_half = rpt // 2
    out_type = [jax.ShapeDtypeStruct((_NC, n_pad, d), jnp.float32)]
    scratch = [
        pltpu.VMEM((rpt_half, _LANES), jnp.int32),
        pltpu.VMEM((rpt_half, _LANES), jnp.int32),
        pltpu.VMEM((_LANES, d), jnp.float32),
        pltpu.VMEM((_LANES, d), jnp.float32),
        pltpu.VMEM_SHARED((n_pad, d), jnp.float32),
        pltpu.SemaphoreType.DMA,
        pltpu.SemaphoreType.DMA,
        pltpu.SemaphoreType.DMA,
        pltpu.SemaphoreType.DMA,
    ]
    rows_per_tile = n_pad // _NS

    def body(u_hbm, src_hbm, dst_hbm, acc_out, src_v, dst_v, rows0_v,
             rows1_v, acc_sh, sem0, sem1, ssem0, ssem1):
        c = lax.axis_index("c")
        s = lax.axis_index("s")
        wid = s * _NC + c

        _fill2d(rows0_v, _LANES, d, 0.0)
        base = s * rows_per_tile
        off = 0
        while off < rows_per_tile:
            step = min(_LANES, rows_per_tile - off)
            pltpu.sync_copy(rows0_v.at[pl.ds(0, step)],
                            acc_sh.at[pl.ds(base + off, step)])
            off += step
        plsc.subcore_barrier()

        def pair_fn(p, carry):
            j0 = 2 * p
            j1 = j0 + 1

            @pl.when(p > 0)
            def _():
                pltpu.make_async_copy(rows0_v, acc_sh.at[dst_v.at[j0]],
                                      ssem0).wait()
                pltpu.make_async_copy(rows1_v, acc_sh.at[dst_v.at[j1]],
                                      ssem1).wait()

            g0 = pltpu.async_copy(u_hbm.at[src_v.at[j0]], rows0_v, sem0)
            g1 = pltpu.async_copy(u_hbm.at[src_v.at[j1]], rows1_v, sem1)
            g0.wait()
            pltpu.async_copy(rows0_v, acc_sh.at[dst_v.at[j0]], ssem0,
                             add=True)
            g1.wait()
            pltpu.async_copy(rows1_v, acc_sh.at[dst_v.at[j1]], ssem1,
                             add=True)
            return carry

        for half in range(2):
            rbase = wid * rpt + half * rpt_half
            pltpu.sync_copy(src_hbm.at[pl.ds(rbase, rpt_half)], src_v)
            pltpu.sync_copy(dst_hbm.at[pl.ds(rbase, rpt_half)], dst_v)
            lax.fori_loop(0, rpt_half // 2, pair_fn, 0)
            pltpu.make_async_copy(rows0_v, acc_sh.at[dst_v.at[0]],
                                  ssem0).wait()
            pltpu.make_async_copy(rows1_v, acc_sh.at[dst_v.at[1]],
                                  ssem1).wait()

        plsc.subcore_barrier()
        pltpu.sync_copy(acc_sh.at[pl.ds(base, rows_per_tile)],
                        acc_out.at[c].at[pl.ds(base, rows_per_tile)])

    return pl.kernel(body, out_type=out_type, mesh=_sc_mesh(),
                     scratch_types=scratch)


def _tc_pre(x, Wnbr1, Wself1):
    n, d_in = x.shape
    d_h = Wnbr1.shape[1]

    def body(x_ref, wn_ref, ws_ref, u_ref, s_ref):
        xb = x_ref[...]
        u_ref[...] = jnp.dot(xb, wn_ref[...], preferred_element_type=jnp.float32,
                 precision=lax.Precision.HIGHEST)
        s_ref[...] = jnp.dot(xb, ws_ref[...], preferred_element_type=jnp.float32,
                 precision=lax.Precision.HIGHEST)

    return pl.pallas_call(
        body,
        grid=(n // _BLK,),
        in_specs=[pl.BlockSpec((_BLK, d_in), lambda i: (i, 0)),
                  pl.BlockSpec((d_in, d_h), lambda i: (0, 0)),
                  pl.BlockSpec((d_in, d_h), lambda i: (0, 0))],
        out_specs=[pl.BlockSpec((_BLK, d_h), lambda i: (i, 0)),
                   pl.BlockSpec((_BLK, d_h), lambda i: (i, 0))],
        out_shape=[jax.ShapeDtypeStruct((n, d_h), jnp.float32),
                   jax.ShapeDtypeStruct((n, d_h), jnp.float32)],
    )(x, Wnbr1, Wself1)


def _tc_mid(s1, acc1, cnt, b1, Wnbr2, Wself2, b2):
    n, d_h = s1.shape
    n_pad = acc1.shape[1]
    d_emb = Wnbr2.shape[1]

    def body(s1_ref, acc_ref, cnt_ref, b1_ref, wn_ref, ws_ref, b2_ref,
             vhs_ref):
        cnt3 = cnt_ref[...]
        deg = jnp.maximum(cnt3[0, :, :1] + cnt3[1, :, :1], 1.0)
        agg = (acc_ref[0] + acc_ref[1]) / deg
        h = jnp.maximum(s1_ref[...] + agg + b1_ref[...], 0.0)
        v = jnp.dot(h, wn_ref[...], preferred_element_type=jnp.float32,
                 precision=lax.Precision.HIGHEST)
        hs = jnp.dot(h, ws_ref[...],
                     preferred_element_type=jnp.float32,
                 precision=lax.Precision.HIGHEST) + b2_ref[...]
        vhs_ref[...] = jnp.concatenate([v, hs], axis=1)

    return pl.pallas_call(
        body,
        grid=(n // _BLK,),
        in_specs=[pl.BlockSpec((_BLK, d_h), lambda i: (i, 0)),
                  pl.BlockSpec((_NC, _BLK, d_h), lambda i: (0, i, 0)),
                  pl.BlockSpec((_NC, _BLK, _LANES), lambda i: (0, i, 0)),
                  pl.BlockSpec((1, d_h), lambda i: (0, 0)),
                  pl.BlockSpec((d_h, d_emb), lambda i: (0, 0)),
                  pl.BlockSpec((d_h, d_emb), lambda i: (0, 0)),
                  pl.BlockSpec((1, d_emb), lambda i: (0, 0))],
        out_specs=[pl.BlockSpec((_BLK, 2 * d_emb), lambda i: (i, 0))],
        out_shape=[jax.ShapeDtypeStruct((n, 2 * d_emb), jnp.float32)],
    )(s1, acc1, cnt, b1, Wnbr2, Wself2, b2)


def _tc_rep(vhs, acc2, cnt, Wd, bd):
    n = vhs.shape[0]
    d_emb = vhs.shape[1] // 2
    d_dec = Wd.shape[1]

    def body(vhs_ref, acc_ref, cnt_ref, wd_ref, bd_ref, rep_ref, st_ref, sacc):
        i = pl.program_id(0)
        cnt3 = cnt_ref[...]
        deg = jnp.maximum(cnt3[0, :, :1] + cnt3[1, :, :1], 1.0)
        hs = vhs_ref[...][:, d_emb:]
        aggv = (acc_ref[0] + acc_ref[1])[:, :d_emb]
        rep = hs + aggv / deg
        rep_ref[...] = rep
        hd = jnp.dot(rep, wd_ref[...],
                     preferred_element_type=jnp.float32,
                 precision=lax.Precision.HIGHEST) + bd_ref[...]

        @pl.when(i == 0)
        def _():
            sacc[...] = jnp.zeros_like(sacc)

        sacc[0:1, :] += jnp.sum(hd, axis=0, keepdims=True)
        sacc[1:2, :] += jnp.sum(hd * hd, axis=0, keepdims=True)
        st_ref[...] = sacc[...]

    return pl.pallas_call(
        body,
        grid=(n // _BLK,),
        in_specs=[pl.BlockSpec((_BLK, 2 * d_emb), lambda i: (i, 0)),
                  pl.BlockSpec((_NC, _BLK, 2 * d_emb), lambda i: (0, i, 0)),
                  pl.BlockSpec((_NC, _BLK, _LANES), lambda i: (0, i, 0)),
                  pl.BlockSpec((d_emb, d_dec), lambda i: (0, 0)),
                  pl.BlockSpec((1, d_dec), lambda i: (0, 0))],
        out_specs=[pl.BlockSpec((_BLK, d_emb), lambda i: (i, 0)),
                   pl.BlockSpec((2, d_dec), lambda i: (0, 0))],
        out_shape=[jax.ShapeDtypeStruct((n, d_emb), jnp.float32),
                   jax.ShapeDtypeStruct((2, d_dec), jnp.float32)],
        scratch_shapes=[pltpu.VMEM((2, d_dec), jnp.float32)],
    )(vhs, acc2, cnt, Wd, bd)


def _tc_heads(rep, stats, Wd, bd, gamma, beta, Wm, bm, Wdi, bdi, Wpi, bpi):
    n, d_emb = rep.shape
    d_dec = Wd.shape[1]
    g = Wm.shape[1]
    n_f = float(n)
    blk = 1000

    def body(rep_ref, st_ref, wd_ref, bd_ref, g_ref, be_ref, wm_ref, bm_ref,
             wdi_ref, bdi_ref, wpi_ref, bpi_ref, mean_ref, disp_ref, pi_ref):
        rep = rep_ref[...]
        hd = jnp.dot(rep, wd_ref[...],
                     preferred_element_type=jnp.float32,
                 precision=lax.Precision.HIGHEST) + bd_ref[...]
        st = st_ref[...]
        mu = st[0:1, :] / n_f
        var = st[1:2, :] / n_f - mu * mu
        inv = lax.rsqrt(var + 1e-5)
        hidden = jnp.maximum((hd - mu) * inv * g_ref[...] + be_ref[...], 0.0)
        m = jnp.dot(hidden, wm_ref[...],
                    preferred_element_type=jnp.float32) + bm_ref[...]
        mean_ref[...] = jnp.clip(jnp.exp(m), 1e-5, 1e6)
        dv = jnp.dot(hidden, wdi_ref[...],
                     preferred_element_type=jnp.float32) + bdi_ref[...]
        sp = jnp.maximum(dv, 0.0) + jnp.log(1.0 + jnp.exp(-jnp.abs(dv)))
        disp_ref[...] = jnp.clip(sp, 1e-4, 1e4)
        pv = jnp.dot(hidden, wpi_ref[...],
                     preferred_element_type=jnp.float32) + bpi_ref[...]
        pi_ref[...] = 1.0 / (1.0 + jnp.exp(-pv))

    return pl.pallas_call(
        body,
        grid=(n // blk,),
        in_specs=[pl.BlockSpec((blk, d_emb), lambda i: (i, 0)),
                  pl.BlockSpec((2, d_dec), lambda i: (0, 0)),
                  pl.BlockSpec((d_emb, d_dec), lambda i: (0, 0)),
                  pl.BlockSpec((1, d_dec), lambda i: (0, 0)),
                  pl.BlockSpec((1, d_dec), lambda i: (0, 0)),
                  pl.BlockSpec((1, d_dec), lambda i: (0, 0)),
                  pl.BlockSpec((d_dec, g), lambda i: (0, 0)),
                  pl.BlockSpec((1, g), lambda i: (0, 0)),
                  pl.BlockSpec((d_dec, g), lambda i: (0, 0)),
                  pl.BlockSpec((1, g), lambda i: (0, 0)),
                  pl.BlockSpec((d_dec, g), lambda i: (0, 0)),
                  pl.BlockSpec((1, g), lambda i: (0, 0))],
        out_specs=[pl.BlockSpec((blk, g), lambda i: (i, 0)),
                   pl.BlockSpec((blk, g), lambda i: (i, 0)),
                   pl.BlockSpec((blk, g), lambda i: (i, 0))],
        out_shape=[jax.ShapeDtypeStruct((n, g), jnp.float32),
                   jax.ShapeDtypeStruct((n, g), jnp.float32),
                   jax.ShapeDtypeStruct((n, g), jnp.float32)],
    )(rep, stats, Wd, bd, gamma, beta, Wm, bm, Wdi, bdi, Wpi, bpi)


def kernel(x, edge_index, Wself1, Wnbr1, b1, Wself2, Wnbr2, b2, Wd, bd,
           gamma, beta, Wm, bm, Wdi, bdi, Wpi, bpi):
    n = x.shape[0]
    e = edge_index.shape[1]
    d_h = Wself1.shape[1]
    d_emb = Wself2.shape[1]

    rpt = -(-e // (_LANES * _NC * _NS))
    rpt = (rpt + 15) // 16 * 16
    rpad = rpt * _NC * _NS
    epad = rpad * _LANES
    n_pad = (n + 128) // 128 * 128

    src = jnp.concatenate(
        [edge_index[0], jnp.zeros((epad - e,), jnp.int32)]).reshape(rpad, _LANES)
    dst = jnp.concatenate(
        [edge_index[1], jnp.full((epad - e,), n, jnp.int32)]).reshape(rpad, _LANES)

    cnt = _sc_count(n_pad, rpt)(dst)
    if isinstance(cnt, (list, tuple)):
        cnt = cnt[0]
    u, s1 = _tc_pre(x, Wnbr1, Wself1)
    acc1 = _sc_segsum(n_pad, d_h, rpt)(u, src, dst)
    if isinstance(acc1, (list, tuple)):
        acc1 = acc1[0]
    vhs = _tc_mid(s1, acc1, cnt, b1.reshape(1, -1), Wnbr2, Wself2,
                  b2.reshape(1, -1))
    if isinstance(vhs, (list, tuple)):
        vhs = vhs[0]
    acc2 = _sc_segsum(n_pad, 2 * d_emb, rpt)(vhs, src, dst)
    if isinstance(acc2, (list, tuple)):
        acc2 = acc2[0]
    rep, stats = _tc_rep(vhs, acc2, cnt, Wd, bd.reshape(1, -1))
    mean, disp, pi = _tc_heads(rep, stats, Wd, bd.reshape(1, -1),
                               gamma.reshape(1, -1), beta.reshape(1, -1),
                               Wm, bm.reshape(1, -1), Wdi, bdi.reshape(1, -1),
                               Wpi, bpi.reshape(1, -1))
    return (mean, disp, pi, rep)
